# baseline (speedup 1.0000x reference)
import functools

import jax
import jax.numpy as jnp
from jax import lax
from jax.experimental import pallas as pl
from jax.experimental.pallas import tpu as pltpu
from jax.experimental.pallas import tpu_sc as plsc

_N = 10000
_E = 320000
_D = 128
_G = 64

_NC = 2
_NS = 16
_NW = _NC * _NS
_EP = _E // _NW
_C = 40
_NCH = _EP // _C
_NBUF = 5
_CD = 80
_NCHD = _EP // _CD
_RT = 632
_NP = _NS * _RT
_RTD = 640
_NPD = _NS * _RTD

_BS = 1000
_NB = _N // _BS


def _sc_mesh():
    return plsc.VectorSubcoreMesh(
        core_axis_name="c", subcore_axis_name="s",
        num_cores=_NC, num_subcores=_NS)


def _sc_degree(dst3):
    @functools.partial(
        pl.kernel, mesh=_sc_mesh(),
        out_type=jax.ShapeDtypeStruct((_NC, _NS, _RTD), jnp.float32),
        scratch_types=[
            pltpu.VMEM((_NCHD, _CD), jnp.int32),
            pltpu.VMEM((_CD,), jnp.float32),
            pltpu.VMEM((_RTD,), jnp.float32),
            pltpu.VMEM_SHARED((_NPD,), jnp.float32),
        ],
    )
    def k(dst_hbm, out_hbm, didx, ones_v, zb, acc):
        cid = lax.axis_index("c")
        sid = lax.axis_index("s")
        wid = sid * _NC + cid
        pltpu.sync_copy(dst_hbm.at[wid], didx)

        def fill_ones(i, _):
            ones_v[pl.ds(i * 16, 16)] = jnp.ones((16,), jnp.float32)
            return 0
        lax.fori_loop(0, _CD // 16, fill_ones, 0)

        def fill_zeros(i, _):
            zb[pl.ds(i * 16, 16)] = jnp.zeros((16,), jnp.float32)
            return 0
        lax.fori_loop(0, _RTD // 16, fill_zeros, 0)

        base = sid * _RTD
        pltpu.sync_copy(zb, acc.at[pl.ds(base, _RTD)])
        plsc.subcore_barrier()

        def step(j, _):
            pltpu.sync_copy(ones_v, acc.at[didx.at[j]], add=True)
            return 0
        lax.fori_loop(0, _NCHD, step, 0)

        plsc.subcore_barrier()
        pltpu.sync_copy(acc.at[pl.ds(base, _RTD)], out_hbm.at[cid, sid])

    return k(dst3).reshape(_NC, _NPD)


def _sc_propagate(u, src2, dst2):
    @functools.partial(
        pl.kernel, mesh=_sc_mesh(),
        out_type=jax.ShapeDtypeStruct((_NC, _NS, _RT, _D), jnp.float32),
        scratch_types=[
            pltpu.VMEM((_EP,), jnp.int32),
            pltpu.VMEM((_EP,), jnp.int32),
            pltpu.VMEM_SHARED((_NP, _D), jnp.float32),
        ] + [pltpu.VMEM((_C, _D), jnp.float32) for _ in range(_NBUF)]
          + [pltpu.SemaphoreType.DMA for _ in range(2 * _NBUF)],
    )
    def k(u_hbm, src_hbm, dst_hbm, out_hbm, sidx, didx, acc, *bufs_sems):
        rows = bufs_sems[:_NBUF]
        sems = bufs_sems[_NBUF:2 * _NBUF]
        ssem = bufs_sems[2 * _NBUF:]
        cid = lax.axis_index("c")
        sid = lax.axis_index("s")
        wid = sid * _NC + cid
        pltpu.sync_copy(src_hbm.at[wid], sidx)
        pltpu.sync_copy(dst_hbm.at[wid], didx)

        def gather(j, b):
            return pltpu.make_async_copy(
                u_hbm.at[sidx.at[pl.ds(j * _C, _C)]], rows[b], sems[b])

        def scat(j, b):
            return pltpu.make_async_copy(
                rows[b], acc.at[didx.at[pl.ds(j * _C, _C)]], ssem[b])

        def zr(i, _):
            for kk in range(_D // 16):
                rows[0][i, pl.ds(kk * 16, 16)] = jnp.zeros((16,), jnp.float32)
            return 0
        lax.fori_loop(0, _C, zr, 0)

        base = sid * _RT

        def zslice(t, _):
            pltpu.sync_copy(rows[0], acc.at[pl.ds(base + t * _C, _C)])
            return 0
        lax.fori_loop(0, _RT // _C, zslice, 0)
        _rem = _RT % _C
        if _rem:
            pltpu.sync_copy(rows[0].at[pl.ds(0, _rem)],
                            acc.at[pl.ds(base + (_RT // _C) * _C, _rem)])
        plsc.subcore_barrier()

        for b in range(_NBUF - 1):
            gather(b, b).start()

        gather(0, 0).wait()
        scat(0, 0).start(add=True)
        gather(_NBUF - 1, _NBUF - 1).start()
        for j in range(1, _NBUF):
            gather(j, j).wait()
            scat(j, j).start(add=True)
            bp = (j - 1) % _NBUF
            scat(j - 1, bp).wait()
            gather(j + _NBUF - 1, bp).start()

        def step(t, _):
            j0 = t * _NBUF
            for b in range(_NBUF):
                j = j0 + b
                gather(j, b).wait()
                scat(j, b).start(add=True)
                bp = (b - 1) % _NBUF
                scat(j - 1, bp).wait()
                gather(j + _NBUF - 1, bp).start()
            return 0
        lax.fori_loop(1, _NCH // _NBUF - 1, step, 0)

        jl = _NCH - _NBUF
        gather(jl, jl % _NBUF).wait()
        scat(jl, jl % _NBUF).start(add=True)
        bp = (jl - 1) % _NBUF
        scat(jl - 1, bp).wait()
        gather(jl + _NBUF - 1, bp).start()
        for j in range(jl + 1, _NCH):
            b = j % _NBUF
            gather(j, b).wait()
            scat(j, b).start(add=True)
        for j in range(jl, _NCH):
            scat(j, j % _NBUF).wait()

        plsc.subcore_barrier()
        pltpu.sync_copy(acc.at[pl.ds(base, _RT)], out_hbm.at[cid, sid])

    return k(u, src2, dst2).reshape(_NC, _NP, _D)


def _prep_body(x_ref, w_ref, dp_ref, u_ref, inv_ref):
    deg = jnp.sum(dp_ref[...], axis=1, keepdims=True) + 1.0
    inv = lax.rsqrt(deg)
    inv_ref[...] = inv
    u_ref[...] = jnp.dot(x_ref[...], w_ref[...],
                         preferred_element_type=jnp.float32) * inv


def _tc_prep(x, W, degp_t):
    return pl.pallas_call(
        _prep_body,
        grid=(_NB,),
        in_specs=[
            pl.BlockSpec((_BS, _D), lambda i: (i, 0)),
            pl.BlockSpec((_D, _D), lambda i: (0, 0)),
            pl.BlockSpec((_BS, _NC), lambda i: (i, 0)),
        ],
        out_specs=[
            pl.BlockSpec((_BS, _D), lambda i: (i, 0)),
            pl.BlockSpec((_BS, 1), lambda i: (i, 0)),
        ],
        out_shape=[
            jax.ShapeDtypeStruct((_N, _D), jnp.float32),
            jax.ShapeDtypeStruct((_N, 1), jnp.float32),
        ],
    )(x, W, degp_t)


def _stats_body(h_ref, u_ref, inv_ref, out_ref):
    i = pl.program_id(0)
    z = (h_ref[0] + h_ref[1] + u_ref[...]) * inv_ref[...]
    s1 = jnp.sum(z, axis=0, keepdims=True)
    s2 = jnp.sum(z * z, axis=0, keepdims=True)
    blk = jnp.concatenate([s1, s2], axis=0)

    @pl.when(i == 0)
    def _():
        out_ref[...] = blk

    @pl.when(i != 0)
    def _():
        out_ref[...] += blk


def _tc_stats(h, u, inv):
    return pl.pallas_call(
        _stats_body,
        grid=(_NB,),
        in_specs=[
            pl.BlockSpec((_NC, _BS, _D), lambda i: (0, i, 0)),
            pl.BlockSpec((_BS, _D), lambda i: (i, 0)),
            pl.BlockSpec((_BS, 1), lambda i: (i, 0)),
        ],
        out_specs=pl.BlockSpec((2, _D), lambda i: (0, 0)),
        out_shape=jax.ShapeDtypeStruct((2, _D), jnp.float32),
    )(h, u, inv)


def _bn_relu(h_ref, u_ref, inv_ref, st_ref, g_ref, be_ref):
    z = (h_ref[0] + h_ref[1] + u_ref[...]) * inv_ref[...]
    mean = st_ref[0:1] * (1.0 / _N)
    var = st_ref[1:2] * (1.0 / _N) - mean * mean
    a = lax.rsqrt(var + 1e-5) * g_ref[...]
    return jnp.maximum((z - mean) * a + be_ref[...], 0.0)


def _norm_mm_body(h_ref, u_ref, inv_ref, st_ref, g_ref, be_ref, w_ref,
                  out_ref):
    y = _bn_relu(h_ref, u_ref, inv_ref, st_ref, g_ref, be_ref)
    out_ref[...] = jnp.dot(y, w_ref[...],
                           preferred_element_type=jnp.float32) * inv_ref[...]


def _tc_norm_mm(h, u, inv, st, gamma, beta, Wn):
    return pl.pallas_call(
        _norm_mm_body,
        grid=(_NB,),
        in_specs=[
            pl.BlockSpec((_NC, _BS, _D), lambda i: (0, i, 0)),
            pl.BlockSpec((_BS, _D), lambda i: (i, 0)),
            pl.BlockSpec((_BS, 1), lambda i: (i, 0)),
            pl.BlockSpec((2, _D), lambda i: (0, 0)),
            pl.BlockSpec((1, _D), lambda i: (0, 0)),
            pl.BlockSpec((1, _D), lambda i: (0, 0)),
            pl.BlockSpec((_D, _D), lambda i: (0, 0)),
        ],
        out_specs=pl.BlockSpec((_BS, _D), lambda i: (i, 0)),
        out_shape=jax.ShapeDtypeStruct((_N, _D), jnp.float32),
    )(h, u, inv, st, gamma.reshape(1, _D), beta.reshape(1, _D), Wn)


def _pool_body(h_ref, u_ref, inv_ref, st_ref, g_ref, be_ref, b_ref, w_ref,
               bo_ref, out_ref, acc_ref):
    i = pl.program_id(0)
    y = _bn_relu(h_ref, u_ref, inv_ref, st_ref, g_ref, be_ref)
    seg = lax.broadcasted_iota(jnp.int32, (_BS, _G), 1)
    onehot = (b_ref[...] == seg).astype(jnp.float32)
    part = lax.dot_general(onehot, y, (((0,), (0,)), ((), ())),
                           preferred_element_type=jnp.float32)

    @pl.when(i == 0)
    def _():
        acc_ref[...] = part

    @pl.when(i != 0)
    def _():
        acc_ref[...] += part

    @pl.when(i == _NB - 1)
    def _():
        out_ref[...] = jnp.dot(acc_ref[...], w_ref[...],
                               preferred_element_type=jnp.float32) + bo_ref[...]


def _tc_pool(h, u, inv, st, gamma, beta, batch, Wout, bout):
    return pl.pallas_call(
        _pool_body,
        grid=(_NB,),
        in_specs=[
            pl.BlockSpec((_NC, _BS, _D), lambda i: (0, i, 0)),
            pl.BlockSpec((_BS, _D), lambda i: (i, 0)),
            pl.BlockSpec((_BS, 1), lambda i: (i, 0)),
            pl.BlockSpec((2, _D), lambda i: (0, 0)),
            pl.BlockSpec((1, _D), lambda i: (0, 0)),
            pl.BlockSpec((1, _D), lambda i: (0, 0)),
            pl.BlockSpec((_BS, 1), lambda i: (i, 0)),
            pl.BlockSpec((_D, _D), lambda i: (0, 0)),
            pl.BlockSpec((1, _D), lambda i: (0, 0)),
        ],
        out_specs=pl.BlockSpec((_G, _D), lambda i: (0, 0)),
        out_shape=jax.ShapeDtypeStruct((_G, _D), jnp.float32),
        scratch_shapes=[pltpu.VMEM((_G, _D), jnp.float32)],
    )(h, u, inv, st, gamma.reshape(1, _D), beta.reshape(1, _D),
      batch, Wout, bout.reshape(1, _D))


def kernel(x, edge_index, batch, W1, b1, gamma1, beta1, W2, b2, gamma2,
           beta2, W3, b3, gamma3, beta3, Wout, bout):
    x = x.astype(jnp.float32)
    src2 = edge_index[0].reshape(_NW, _EP)
    dst2 = edge_index[1].reshape(_NW, _EP)
    dst3d = edge_index[1].reshape(_NW, _NCHD, _CD)

    degp = _sc_degree(dst3d)
    u, inv = _tc_prep(x, W1, degp.T)

    for gamma, beta, Wn in ((gamma1, beta1, W2), (gamma2, beta2, W3)):
        h = _sc_propagate(u, src2, dst2)
        st = _tc_stats(h, u, inv)
        u = _tc_norm_mm(h, u, inv, st, gamma, beta, Wn)

    h = _sc_propagate(u, src2, dst2)
    st = _tc_stats(h, u, inv)
    return _tc_pool(h, u, inv, st, gamma3, beta3, batch.reshape(_N, 1),
                    Wout, bout.reshape(1, _D))


# speedup vs baseline: 1.0275x; 1.0275x over previous
import functools

import jax
import jax.numpy as jnp
from jax import lax
from jax.experimental import pallas as pl
from jax.experimental.pallas import tpu as pltpu
from jax.experimental.pallas import tpu_sc as plsc

_N = 10000
_E = 320000
_D = 128
_G = 64

_NC = 2
_NS = 16
_NW = _NC * _NS
_EP = _E // _NW
_C = 40
_NCH = _EP // _C
_NBUF = 5
_CD = 80
_NCHD = _EP // _CD
_RT = 632
_NP = _NS * _RT
_RTD = 640
_NPD = _NS * _RTD

_BS = 1000
_NB = _N // _BS


def _sc_mesh():
    return plsc.VectorSubcoreMesh(
        core_axis_name="c", subcore_axis_name="s",
        num_cores=_NC, num_subcores=_NS)


def _sc_degree(dst3):
    @functools.partial(
        pl.kernel, mesh=_sc_mesh(),
        out_type=jax.ShapeDtypeStruct((_NC, _NS, _RTD), jnp.float32),
        scratch_types=[
            pltpu.VMEM((_NCHD, _CD), jnp.int32),
            pltpu.VMEM((_CD,), jnp.float32),
            pltpu.VMEM((_RTD,), jnp.float32),
            pltpu.VMEM_SHARED((_NPD,), jnp.float32),
        ],
    )
    def k(dst_hbm, out_hbm, didx, ones_v, zb, acc):
        cid = lax.axis_index("c")
        sid = lax.axis_index("s")
        wid = sid * _NC + cid
        pltpu.sync_copy(dst_hbm.at[wid], didx)

        def fill_ones(i, _):
            ones_v[pl.ds(i * 16, 16)] = jnp.ones((16,), jnp.float32)
            return 0
        lax.fori_loop(0, _CD // 16, fill_ones, 0)

        def fill_zeros(i, _):
            zb[pl.ds(i * 16, 16)] = jnp.zeros((16,), jnp.float32)
            return 0
        lax.fori_loop(0, _RTD // 16, fill_zeros, 0)

        base = sid * _RTD
        pltpu.sync_copy(zb, acc.at[pl.ds(base, _RTD)])
        plsc.subcore_barrier()

        def step(j, _):
            pltpu.sync_copy(ones_v, acc.at[didx.at[j]], add=True)
            return 0
        lax.fori_loop(0, _NCHD, step, 0)

        plsc.subcore_barrier()
        pltpu.sync_copy(acc.at[pl.ds(base, _RTD)], out_hbm.at[cid, sid])

    return k(dst3).reshape(_NC, _NPD)


def _sc_propagate(u, src2, dst2):
    @functools.partial(
        pl.kernel, mesh=_sc_mesh(),
        out_type=jax.ShapeDtypeStruct((_NC, _NS, _RT, _D), jnp.float32),
        scratch_types=[
            pltpu.VMEM((_EP,), jnp.int32),
            pltpu.VMEM((_EP,), jnp.int32),
            pltpu.VMEM_SHARED((_NP, _D), jnp.float32),
        ] + [pltpu.VMEM((_C, _D), jnp.float32) for _ in range(_NBUF)]
          + [pltpu.SemaphoreType.DMA for _ in range(2 * _NBUF)],
    )
    def k(u_hbm, src_hbm, dst_hbm, out_hbm, sidx, didx, acc, *bufs_sems):
        rows = bufs_sems[:_NBUF]
        sems = bufs_sems[_NBUF:2 * _NBUF]
        ssem = bufs_sems[2 * _NBUF:]
        cid = lax.axis_index("c")
        sid = lax.axis_index("s")
        wid = sid * _NC + cid
        pltpu.sync_copy(src_hbm.at[wid], sidx)
        pltpu.sync_copy(dst_hbm.at[wid], didx)

        def gather(j, b):
            return pltpu.make_async_copy(
                u_hbm.at[sidx.at[pl.ds(j * _C, _C)]], rows[b], sems[b])

        def scat(j, b):
            return pltpu.make_async_copy(
                rows[b], acc.at[didx.at[pl.ds(j * _C, _C)]], ssem[b])

        def zr(i, _):
            for kk in range(_D // 16):
                rows[0][i, pl.ds(kk * 16, 16)] = jnp.zeros((16,), jnp.float32)
            return 0
        lax.fori_loop(0, _C, zr, 0)

        base = sid * _RT
        _zc = (_C // 8) * 8

        def zslice(t, _):
            pltpu.sync_copy(rows[0].at[pl.ds(0, _zc)],
                            acc.at[pl.ds(base + t * _zc, _zc)])
            return 0
        lax.fori_loop(0, _RT // _zc, zslice, 0)
        _rem = _RT % _zc
        if _rem:
            pltpu.sync_copy(rows[0].at[pl.ds(0, _rem)],
                            acc.at[pl.ds(base + (_RT // _zc) * _zc, _rem)])
        plsc.subcore_barrier()

        for b in range(_NBUF - 1):
            gather(b, b).start()

        gather(0, 0).wait()
        scat(0, 0).start(add=True)
        gather(_NBUF - 1, _NBUF - 1).start()
        for j in range(1, _NBUF):
            gather(j, j).wait()
            scat(j, j).start(add=True)
            bp = (j - 1) % _NBUF
            scat(j - 1, bp).wait()
            gather(j + _NBUF - 1, bp).start()

        def step(t, _):
            j0 = t * _NBUF
            for b in range(_NBUF):
                j = j0 + b
                gather(j, b).wait()
                scat(j, b).start(add=True)
                bp = (b - 1) % _NBUF
                scat(j - 1, bp).wait()
                gather(j + _NBUF - 1, bp).start()
            return 0
        lax.fori_loop(1, _NCH // _NBUF - 1, step, 0)

        jl = _NCH - _NBUF
        gather(jl, jl % _NBUF).wait()
        scat(jl, jl % _NBUF).start(add=True)
        bp = (jl - 1) % _NBUF
        scat(jl - 1, bp).wait()
        gather(jl + _NBUF - 1, bp).start()
        for j in range(jl + 1, _NCH):
            b = j % _NBUF
            gather(j, b).wait()
            scat(j, b).start(add=True)
        for j in range(jl, _NCH):
            scat(j, j % _NBUF).wait()

        plsc.subcore_barrier()
        pltpu.sync_copy(acc.at[pl.ds(base, _RT)], out_hbm.at[cid, sid])

    return k(u, src2, dst2).reshape(_NC, _NP, _D)


def _prep_body(x_ref, w_ref, dp_ref, u_ref, inv_ref):
    deg = jnp.sum(dp_ref[...], axis=1, keepdims=True) + 1.0
    inv = lax.rsqrt(deg)
    inv_ref[...] = inv
    u_ref[...] = jnp.dot(x_ref[...], w_ref[...],
                         preferred_element_type=jnp.float32) * inv


def _tc_prep(x, W, degp_t):
    return pl.pallas_call(
        _prep_body,
        grid=(_NB,),
        in_specs=[
            pl.BlockSpec((_BS, _D), lambda i: (i, 0)),
            pl.BlockSpec((_D, _D), lambda i: (0, 0)),
            pl.BlockSpec((_BS, _NC), lambda i: (i, 0)),
        ],
        out_specs=[
            pl.BlockSpec((_BS, _D), lambda i: (i, 0)),
            pl.BlockSpec((_BS, 1), lambda i: (i, 0)),
        ],
        out_shape=[
            jax.ShapeDtypeStruct((_N, _D), jnp.float32),
            jax.ShapeDtypeStruct((_N, 1), jnp.float32),
        ],
    )(x, W, degp_t)


def _z_stats_phase(h_ref, u_ref, inv_ref, z_ref, st_ref, i):
    z = (h_ref[0] + h_ref[1] + u_ref[...]) * inv_ref[...]
    z_ref[pl.ds(i * _BS, _BS)] = z
    s1 = jnp.sum(z, axis=0, keepdims=True)
    s2 = jnp.sum(z * z, axis=0, keepdims=True)
    blk = jnp.concatenate([s1, s2], axis=0)

    @pl.when(i == 0)
    def _():
        st_ref[...] = blk

    @pl.when(i != 0)
    def _():
        st_ref[...] += blk


def _bn_relu_phase(z_ref, st_ref, g_ref, be_ref, i):
    z = z_ref[pl.ds(i * _BS, _BS)]
    mean = st_ref[0:1] * (1.0 / _N)
    var = st_ref[1:2] * (1.0 / _N) - mean * mean
    a = lax.rsqrt(var + 1e-5) * g_ref[...]
    return jnp.maximum((z - mean) * a + be_ref[...], 0.0)


def _layer_body(h_ref, u_ref, inv_ref, g_ref, be_ref, w_ref, out_ref,
                z_ref, st_ref):
    p = pl.program_id(0)
    i = pl.program_id(1)

    @pl.when(p == 0)
    def _():
        _z_stats_phase(h_ref, u_ref, inv_ref, z_ref, st_ref, i)

    @pl.when(p == 1)
    def _():
        y = _bn_relu_phase(z_ref, st_ref, g_ref, be_ref, i)
        out_ref[...] = jnp.dot(
            y, w_ref[...], preferred_element_type=jnp.float32) * inv_ref[...]


def _tc_layer(h, u, inv, gamma, beta, Wn):
    return pl.pallas_call(
        _layer_body,
        grid=(2, _NB),
        in_specs=[
            pl.BlockSpec((_NC, _BS, _D), lambda p, i: (0, i * (1 - p), 0)),
            pl.BlockSpec((_BS, _D), lambda p, i: (i * (1 - p), 0)),
            pl.BlockSpec((_BS, 1), lambda p, i: (i, 0)),
            pl.BlockSpec((1, _D), lambda p, i: (0, 0)),
            pl.BlockSpec((1, _D), lambda p, i: (0, 0)),
            pl.BlockSpec((_D, _D), lambda p, i: (0, 0)),
        ],
        out_specs=pl.BlockSpec((_BS, _D), lambda p, i: (i * p, 0)),
        out_shape=jax.ShapeDtypeStruct((_N, _D), jnp.float32),
        scratch_shapes=[
            pltpu.VMEM((_N, _D), jnp.float32),
            pltpu.VMEM((2, _D), jnp.float32),
        ],
    )(h, u, inv, gamma.reshape(1, _D), beta.reshape(1, _D), Wn)


def _poolf_body(h_ref, u_ref, inv_ref, g_ref, be_ref, b_ref, w_ref, bo_ref,
                out_ref, z_ref, st_ref, acc_ref):
    p = pl.program_id(0)
    i = pl.program_id(1)

    @pl.when(p == 0)
    def _():
        _z_stats_phase(h_ref, u_ref, inv_ref, z_ref, st_ref, i)

    @pl.when(p == 1)
    def _():
        y = _bn_relu_phase(z_ref, st_ref, g_ref, be_ref, i)
        seg = lax.broadcasted_iota(jnp.int32, (_BS, _G), 1)
        onehot = (b_ref[...] == seg).astype(jnp.float32)
        part = lax.dot_general(onehot, y, (((0,), (0,)), ((), ())),
                               preferred_element_type=jnp.float32)

        @pl.when(i == 0)
        def _():
            acc_ref[...] = part

        @pl.when(i != 0)
        def _():
            acc_ref[...] += part

        @pl.when(i == _NB - 1)
        def _():
            out_ref[...] = jnp.dot(
                acc_ref[...], w_ref[...],
                preferred_element_type=jnp.float32) + bo_ref[...]


def _tc_poolf(h, u, inv, gamma, beta, batch, Wout, bout):
    return pl.pallas_call(
        _poolf_body,
        grid=(2, _NB),
        in_specs=[
            pl.BlockSpec((_NC, _BS, _D), lambda p, i: (0, i * (1 - p), 0)),
            pl.BlockSpec((_BS, _D), lambda p, i: (i * (1 - p), 0)),
            pl.BlockSpec((_BS, 1), lambda p, i: (i, 0)),
            pl.BlockSpec((1, _D), lambda p, i: (0, 0)),
            pl.BlockSpec((1, _D), lambda p, i: (0, 0)),
            pl.BlockSpec((_BS, 1), lambda p, i: (i, 0)),
            pl.BlockSpec((_D, _D), lambda p, i: (0, 0)),
            pl.BlockSpec((1, _D), lambda p, i: (0, 0)),
        ],
        out_specs=pl.BlockSpec((_G, _D), lambda p, i: (0, 0)),
        out_shape=jax.ShapeDtypeStruct((_G, _D), jnp.float32),
        scratch_shapes=[
            pltpu.VMEM((_N, _D), jnp.float32),
            pltpu.VMEM((2, _D), jnp.float32),
            pltpu.VMEM((_G, _D), jnp.float32),
        ],
    )(h, u, inv, gamma.reshape(1, _D), beta.reshape(1, _D),
      batch, Wout, bout.reshape(1, _D))


def kernel(x, edge_index, batch, W1, b1, gamma1, beta1, W2, b2, gamma2,
           beta2, W3, b3, gamma3, beta3, Wout, bout):
    x = x.astype(jnp.float32)
    src2 = edge_index[0].reshape(_NW, _EP)
    dst2 = edge_index[1].reshape(_NW, _EP)
    dst3d = edge_index[1].reshape(_NW, _NCHD, _CD)

    degp = _sc_degree(dst3d)
    u, inv = _tc_prep(x, W1, degp.T)

    for gamma, beta, Wn in ((gamma1, beta1, W2), (gamma2, beta2, W3)):
        h = _sc_propagate(u, src2, dst2)
        u = _tc_layer(h, u, inv, gamma, beta, Wn)

    h = _sc_propagate(u, src2, dst2)
    return _tc_poolf(h, u, inv, gamma3, beta3, batch.reshape(_N, 1),
                     Wout, bout)


# speedup vs baseline: 1.0688x; 1.0402x over previous
import functools

import jax
import jax.numpy as jnp
from jax import lax
from jax.experimental import pallas as pl
from jax.experimental.pallas import tpu as pltpu
from jax.experimental.pallas import tpu_sc as plsc

_N = 10000
_E = 320000
_D = 128
_G = 64

_NC = 2
_NS = 16
_NW = _NC * _NS
_EP = _E // _NW
_C = 40
_NCH = _EP // _C
_NBUF = 5
_CD = 80
_NCHD = _EP // _CD
_RT = 632
_NP = _NS * _RT
_RTD = 640
_NPD = _NS * _RTD

_BS = 2000
_NB = _N // _BS


def _sc_mesh():
    return plsc.VectorSubcoreMesh(
        core_axis_name="c", subcore_axis_name="s",
        num_cores=_NC, num_subcores=_NS)


def _sc_degree(dst3):
    @functools.partial(
        pl.kernel, mesh=_sc_mesh(),
        out_type=jax.ShapeDtypeStruct((_NC, _NS, _RTD), jnp.float32),
        scratch_types=[
            pltpu.VMEM((_NCHD, _CD), jnp.int32),
            pltpu.VMEM((_CD,), jnp.float32),
            pltpu.VMEM((_RTD,), jnp.float32),
            pltpu.VMEM_SHARED((_NPD,), jnp.float32),
        ],
    )
    def k(dst_hbm, out_hbm, didx, ones_v, zb, acc):
        cid = lax.axis_index("c")
        sid = lax.axis_index("s")
        wid = sid * _NC + cid
        pltpu.sync_copy(dst_hbm.at[wid], didx)

        def fill_ones(i, _):
            ones_v[pl.ds(i * 16, 16)] = jnp.ones((16,), jnp.float32)
            return 0
        lax.fori_loop(0, _CD // 16, fill_ones, 0)

        def fill_zeros(i, _):
            zb[pl.ds(i * 16, 16)] = jnp.zeros((16,), jnp.float32)
            return 0
        lax.fori_loop(0, _RTD // 16, fill_zeros, 0)

        base = sid * _RTD
        pltpu.sync_copy(zb, acc.at[pl.ds(base, _RTD)])
        plsc.subcore_barrier()

        def step(j, _):
            pltpu.sync_copy(ones_v, acc.at[didx.at[j]], add=True)
            return 0
        lax.fori_loop(0, _NCHD, step, 0)

        plsc.subcore_barrier()
        pltpu.sync_copy(acc.at[pl.ds(base, _RTD)], out_hbm.at[cid, sid])

    return k(dst3).reshape(_NC, _NPD)


def _sc_propagate(u, src2, dst2):
    @functools.partial(
        pl.kernel, mesh=_sc_mesh(),
        out_type=jax.ShapeDtypeStruct((_NC, _NS, _RT, _D), jnp.float32),
        scratch_types=[
            pltpu.VMEM((_EP,), jnp.int32),
            pltpu.VMEM((_EP,), jnp.int32),
            pltpu.VMEM_SHARED((_NP, _D), jnp.float32),
        ] + [pltpu.VMEM((_C, _D), jnp.float32) for _ in range(_NBUF)]
          + [pltpu.SemaphoreType.DMA for _ in range(2 * _NBUF)],
    )
    def k(u_hbm, src_hbm, dst_hbm, out_hbm, sidx, didx, acc, *bufs_sems):
        rows = bufs_sems[:_NBUF]
        sems = bufs_sems[_NBUF:2 * _NBUF]
        ssem = bufs_sems[2 * _NBUF:]
        cid = lax.axis_index("c")
        sid = lax.axis_index("s")
        wid = sid * _NC + cid
        pltpu.sync_copy(src_hbm.at[wid], sidx)
        pltpu.sync_copy(dst_hbm.at[wid], didx)

        def gather(j, b):
            return pltpu.make_async_copy(
                u_hbm.at[sidx.at[pl.ds(j * _C, _C)]], rows[b], sems[b])

        def scat(j, b):
            return pltpu.make_async_copy(
                rows[b], acc.at[didx.at[pl.ds(j * _C, _C)]], ssem[b])

        def zr(i, _):
            for kk in range(_D // 16):
                rows[0][i, pl.ds(kk * 16, 16)] = jnp.zeros((16,), jnp.float32)
            return 0
        lax.fori_loop(0, _C, zr, 0)

        base = sid * _RT
        _zc = (_C // 8) * 8

        def zslice(t, _):
            pltpu.sync_copy(rows[0].at[pl.ds(0, _zc)],
                            acc.at[pl.ds(base + t * _zc, _zc)])
            return 0
        lax.fori_loop(0, _RT // _zc, zslice, 0)
        _rem = _RT % _zc
        if _rem:
            pltpu.sync_copy(rows[0].at[pl.ds(0, _rem)],
                            acc.at[pl.ds(base + (_RT // _zc) * _zc, _rem)])
        plsc.subcore_barrier()

        for b in range(_NBUF - 1):
            gather(b, b).start()

        gather(0, 0).wait()
        scat(0, 0).start(add=True)
        gather(_NBUF - 1, _NBUF - 1).start()
        for j in range(1, _NBUF):
            gather(j, j).wait()
            scat(j, j).start(add=True)
            bp = (j - 1) % _NBUF
            scat(j - 1, bp).wait()
            gather(j + _NBUF - 1, bp).start()

        def step(t, _):
            j0 = t * _NBUF
            for b in range(_NBUF):
                j = j0 + b
                gather(j, b).wait()
                scat(j, b).start(add=True)
                bp = (b - 1) % _NBUF
                scat(j - 1, bp).wait()
                gather(j + _NBUF - 1, bp).start()
            return 0
        lax.fori_loop(1, _NCH // _NBUF - 1, step, 0)

        jl = _NCH - _NBUF
        gather(jl, jl % _NBUF).wait()
        scat(jl, jl % _NBUF).start(add=True)
        bp = (jl - 1) % _NBUF
        scat(jl - 1, bp).wait()
        gather(jl + _NBUF - 1, bp).start()
        for j in range(jl + 1, _NCH):
            b = j % _NBUF
            gather(j, b).wait()
            scat(j, b).start(add=True)
        for j in range(jl, _NCH):
            scat(j, j % _NBUF).wait()

        plsc.subcore_barrier()
        pltpu.sync_copy(acc.at[pl.ds(base, _RT)], out_hbm.at[cid, sid])

    return k(u, src2, dst2).reshape(_NC, _NP, _D)


def _prep_body(x_ref, w_ref, dp_ref, u_ref, inv_ref):
    deg = jnp.sum(dp_ref[...], axis=1, keepdims=True) + 1.0
    inv = lax.rsqrt(deg)
    inv_ref[...] = inv
    u_ref[...] = jnp.dot(x_ref[...], w_ref[...],
                         preferred_element_type=jnp.float32) * inv


def _tc_prep(x, W, degp_t):
    return pl.pallas_call(
        _prep_body,
        grid=(_NB,),
        in_specs=[
            pl.BlockSpec((_BS, _D), lambda i: (i, 0)),
            pl.BlockSpec((_D, _D), lambda i: (0, 0)),
            pl.BlockSpec((_BS, _NC), lambda i: (i, 0)),
        ],
        out_specs=[
            pl.BlockSpec((_BS, _D), lambda i: (i, 0)),
            pl.BlockSpec((_BS, 1), lambda i: (i, 0)),
        ],
        out_shape=[
            jax.ShapeDtypeStruct((_N, _D), jnp.float32),
            jax.ShapeDtypeStruct((_N, 1), jnp.float32),
        ],
    )(x, W, degp_t)


def _z_stats_phase(h_ref, u_ref, inv_ref, z_ref, st_ref, i):
    z = (h_ref[0] + h_ref[1] + u_ref[...]) * inv_ref[...]
    z_ref[pl.ds(i * _BS, _BS)] = z
    s1 = jnp.sum(z, axis=0, keepdims=True)
    s2 = jnp.sum(z * z, axis=0, keepdims=True)
    blk = jnp.concatenate([s1, s2], axis=0)

    @pl.when(i == 0)
    def _():
        st_ref[...] = blk

    @pl.when(i != 0)
    def _():
        st_ref[...] += blk


def _bn_relu_phase(z_ref, st_ref, g_ref, be_ref, i):
    z = z_ref[pl.ds(i * _BS, _BS)]
    mean = st_ref[0:1] * (1.0 / _N)
    var = st_ref[1:2] * (1.0 / _N) - mean * mean
    a = lax.rsqrt(var + 1e-5) * g_ref[...]
    return jnp.maximum((z - mean) * a + be_ref[...], 0.0)


def _layer_body(h_ref, u_ref, inv_ref, g_ref, be_ref, w_ref, out_ref,
                z_ref, st_ref):
    p = pl.program_id(0)
    i = pl.program_id(1)

    @pl.when(p == 0)
    def _():
        _z_stats_phase(h_ref, u_ref, inv_ref, z_ref, st_ref, i)

    @pl.when(p == 1)
    def _():
        y = _bn_relu_phase(z_ref, st_ref, g_ref, be_ref, i)
        out_ref[...] = jnp.dot(
            y, w_ref[...], preferred_element_type=jnp.float32) * inv_ref[...]


def _tc_layer(h, u, inv, gamma, beta, Wn):
    return pl.pallas_call(
        _layer_body,
        grid=(2, _NB),
        in_specs=[
            pl.BlockSpec((_NC, _BS, _D), lambda p, i: (0, i * (1 - p), 0)),
            pl.BlockSpec((_BS, _D), lambda p, i: (i * (1 - p), 0)),
            pl.BlockSpec((_BS, 1), lambda p, i: (i, 0)),
            pl.BlockSpec((1, _D), lambda p, i: (0, 0)),
            pl.BlockSpec((1, _D), lambda p, i: (0, 0)),
            pl.BlockSpec((_D, _D), lambda p, i: (0, 0)),
        ],
        out_specs=pl.BlockSpec((_BS, _D), lambda p, i: (i * p, 0)),
        out_shape=jax.ShapeDtypeStruct((_N, _D), jnp.float32),
        scratch_shapes=[
            pltpu.VMEM((_N, _D), jnp.float32),
            pltpu.VMEM((2, _D), jnp.float32),
        ],
    )(h, u, inv, gamma.reshape(1, _D), beta.reshape(1, _D), Wn)


def _poolf_body(h_ref, u_ref, inv_ref, g_ref, be_ref, b_ref, w_ref, bo_ref,
                out_ref, z_ref, st_ref, acc_ref):
    p = pl.program_id(0)
    i = pl.program_id(1)

    @pl.when(p == 0)
    def _():
        _z_stats_phase(h_ref, u_ref, inv_ref, z_ref, st_ref, i)

    @pl.when(p == 1)
    def _():
        y = _bn_relu_phase(z_ref, st_ref, g_ref, be_ref, i)
        seg = lax.broadcasted_iota(jnp.int32, (_BS, _G), 1)
        onehot = (b_ref[...] == seg).astype(jnp.float32)
        part = lax.dot_general(onehot, y, (((0,), (0,)), ((), ())),
                               preferred_element_type=jnp.float32)

        @pl.when(i == 0)
        def _():
            acc_ref[...] = part

        @pl.when(i != 0)
        def _():
            acc_ref[...] += part

        @pl.when(i == _NB - 1)
        def _():
            out_ref[...] = jnp.dot(
                acc_ref[...], w_ref[...],
                preferred_element_type=jnp.float32) + bo_ref[...]


def _tc_poolf(h, u, inv, gamma, beta, batch, Wout, bout):
    return pl.pallas_call(
        _poolf_body,
        grid=(2, _NB),
        in_specs=[
            pl.BlockSpec((_NC, _BS, _D), lambda p, i: (0, i * (1 - p), 0)),
            pl.BlockSpec((_BS, _D), lambda p, i: (i * (1 - p), 0)),
            pl.BlockSpec((_BS, 1), lambda p, i: (i, 0)),
            pl.BlockSpec((1, _D), lambda p, i: (0, 0)),
            pl.BlockSpec((1, _D), lambda p, i: (0, 0)),
            pl.BlockSpec((_BS, 1), lambda p, i: (i, 0)),
            pl.BlockSpec((_D, _D), lambda p, i: (0, 0)),
            pl.BlockSpec((1, _D), lambda p, i: (0, 0)),
        ],
        out_specs=pl.BlockSpec((_G, _D), lambda p, i: (0, 0)),
        out_shape=jax.ShapeDtypeStruct((_G, _D), jnp.float32),
        scratch_shapes=[
            pltpu.VMEM((_N, _D), jnp.float32),
            pltpu.VMEM((2, _D), jnp.float32),
            pltpu.VMEM((_G, _D), jnp.float32),
        ],
    )(h, u, inv, gamma.reshape(1, _D), beta.reshape(1, _D),
      batch, Wout, bout.reshape(1, _D))


def kernel(x, edge_index, batch, W1, b1, gamma1, beta1, W2, b2, gamma2,
           beta2, W3, b3, gamma3, beta3, Wout, bout):
    x = x.astype(jnp.float32)
    src2 = edge_index[0].reshape(_NW, _EP)
    dst2 = edge_index[1].reshape(_NW, _EP)
    dst3d = edge_index[1].reshape(_NW, _NCHD, _CD)

    degp = _sc_degree(dst3d)
    u, inv = _tc_prep(x, W1, degp.T)

    for gamma, beta, Wn in ((gamma1, beta1, W2), (gamma2, beta2, W3)):
        h = _sc_propagate(u, src2, dst2)
        u = _tc_layer(h, u, inv, gamma, beta, Wn)

    h = _sc_propagate(u, src2, dst2)
    return _tc_poolf(h, u, inv, gamma3, beta3, batch.reshape(_N, 1),
                     Wout, bout)


# speedup vs baseline: 1.0907x; 1.0205x over previous
import functools

import jax
import jax.numpy as jnp
from jax import lax
from jax.experimental import pallas as pl
from jax.experimental.pallas import tpu as pltpu
from jax.experimental.pallas import tpu_sc as plsc

_N = 10000
_E = 320000
_D = 128
_G = 64

_NC = 2
_NS = 16
_NW = _NC * _NS
_EP = _E // _NW
_C = 40
_NCH = _EP // _C
_NBUF = 5
_CD = 80
_NCHD = _EP // _CD
_NBD = 5
_RT = 632
_NP = _NS * _RT
_RTD = 640
_NPD = _NS * _RTD

_BS = 2000
_NB = _N // _BS


def _sc_mesh():
    return plsc.VectorSubcoreMesh(
        core_axis_name="c", subcore_axis_name="s",
        num_cores=_NC, num_subcores=_NS)


def _sc_degree(dst3):
    @functools.partial(
        pl.kernel, mesh=_sc_mesh(),
        out_type=jax.ShapeDtypeStruct((_NC, _NS, _RTD), jnp.float32),
        scratch_types=[
            pltpu.VMEM((_NCHD, _CD), jnp.int32),
            pltpu.VMEM((_CD,), jnp.float32),
            pltpu.VMEM((_RTD,), jnp.float32),
            pltpu.VMEM_SHARED((_NPD,), jnp.float32),
        ] + [pltpu.SemaphoreType.DMA for _ in range(_NBD)],
    )
    def k(dst_hbm, out_hbm, didx, ones_v, zb, acc, *sems):
        cid = lax.axis_index("c")
        sid = lax.axis_index("s")
        wid = sid * _NC + cid
        pltpu.sync_copy(dst_hbm.at[wid], didx)

        def fill_ones(i, _):
            ones_v[pl.ds(i * 16, 16)] = jnp.ones((16,), jnp.float32)
            return 0
        lax.fori_loop(0, _CD // 16, fill_ones, 0)

        def fill_zeros(i, _):
            zb[pl.ds(i * 16, 16)] = jnp.zeros((16,), jnp.float32)
            return 0
        lax.fori_loop(0, _RTD // 16, fill_zeros, 0)

        base = sid * _RTD
        pltpu.sync_copy(zb, acc.at[pl.ds(base, _RTD)])
        plsc.subcore_barrier()

        def scat(j, b):
            return pltpu.make_async_copy(ones_v, acc.at[didx.at[j]], sems[b])

        for b in range(_NBD):
            scat(b, b).start(add=True)

        def step(t, _):
            for b in range(_NBD):
                scat(t * _NBD + b - _NBD, b).wait()
                scat(t * _NBD + b, b).start(add=True)
            return 0
        lax.fori_loop(1, _NCHD // _NBD, step, 0)
        for b in range(_NBD):
            scat(_NCHD - _NBD + b, b).wait()

        plsc.subcore_barrier()
        pltpu.sync_copy(acc.at[pl.ds(base, _RTD)], out_hbm.at[cid, sid])

    return k(dst3).reshape(_NC, _NPD)


def _sc_propagate(u, src2, dst2):
    @functools.partial(
        pl.kernel, mesh=_sc_mesh(),
        out_type=jax.ShapeDtypeStruct((_NC, _NS, _RT, _D), jnp.float32),
        scratch_types=[
            pltpu.VMEM((_EP,), jnp.int32),
            pltpu.VMEM((_EP,), jnp.int32),
            pltpu.VMEM_SHARED((_NP, _D), jnp.float32),
        ] + [pltpu.VMEM((_C, _D), jnp.float32) for _ in range(_NBUF)]
          + [pltpu.SemaphoreType.DMA for _ in range(2 * _NBUF)],
    )
    def k(u_hbm, src_hbm, dst_hbm, out_hbm, sidx, didx, acc, *bufs_sems):
        rows = bufs_sems[:_NBUF]
        sems = bufs_sems[_NBUF:2 * _NBUF]
        ssem = bufs_sems[2 * _NBUF:]
        cid = lax.axis_index("c")
        sid = lax.axis_index("s")
        wid = sid * _NC + cid
        pltpu.sync_copy(src_hbm.at[wid], sidx)
        pltpu.sync_copy(dst_hbm.at[wid], didx)

        def gather(j, b):
            return pltpu.make_async_copy(
                u_hbm.at[sidx.at[pl.ds(j * _C, _C)]], rows[b], sems[b])

        def scat(j, b):
            return pltpu.make_async_copy(
                rows[b], acc.at[didx.at[pl.ds(j * _C, _C)]], ssem[b])

        def zr(i, _):
            for kk in range(_D // 16):
                rows[0][i, pl.ds(kk * 16, 16)] = jnp.zeros((16,), jnp.float32)
            return 0
        lax.fori_loop(0, _C, zr, 0)

        base = sid * _RT
        _zc = (_C // 8) * 8

        def zslice(t, _):
            pltpu.sync_copy(rows[0].at[pl.ds(0, _zc)],
                            acc.at[pl.ds(base + t * _zc, _zc)])
            return 0
        lax.fori_loop(0, _RT // _zc, zslice, 0)
        _rem = _RT % _zc
        if _rem:
            pltpu.sync_copy(rows[0].at[pl.ds(0, _rem)],
                            acc.at[pl.ds(base + (_RT // _zc) * _zc, _rem)])
        plsc.subcore_barrier()

        for b in range(_NBUF - 1):
            gather(b, b).start()

        gather(0, 0).wait()
        scat(0, 0).start(add=True)
        gather(_NBUF - 1, _NBUF - 1).start()
        for j in range(1, _NBUF):
            gather(j, j).wait()
            scat(j, j).start(add=True)
            bp = (j - 1) % _NBUF
            scat(j - 1, bp).wait()
            gather(j + _NBUF - 1, bp).start()

        def step(t, _):
            j0 = t * _NBUF
            for b in range(_NBUF):
                j = j0 + b
                gather(j, b).wait()
                scat(j, b).start(add=True)
                bp = (b - 1) % _NBUF
                scat(j - 1, bp).wait()
                gather(j + _NBUF - 1, bp).start()
            return 0
        lax.fori_loop(1, _NCH // _NBUF - 1, step, 0)

        jl = _NCH - _NBUF
        gather(jl, jl % _NBUF).wait()
        scat(jl, jl % _NBUF).start(add=True)
        bp = (jl - 1) % _NBUF
        scat(jl - 1, bp).wait()
        gather(jl + _NBUF - 1, bp).start()
        for j in range(jl + 1, _NCH):
            b = j % _NBUF
            gather(j, b).wait()
            scat(j, b).start(add=True)
        for j in range(jl, _NCH):
            scat(j, j % _NBUF).wait()

        plsc.subcore_barrier()
        pltpu.sync_copy(acc.at[pl.ds(base, _RT)], out_hbm.at[cid, sid])

    return k(u, src2, dst2).reshape(_NC, _NP, _D)


def _prep_body(x_ref, w_ref, dp_ref, u_ref, inv_ref):
    deg = jnp.sum(dp_ref[...], axis=1, keepdims=True) + 1.0
    inv = lax.rsqrt(deg)
    inv_ref[...] = inv
    u_ref[...] = jnp.dot(x_ref[...], w_ref[...],
                         preferred_element_type=jnp.float32) * inv


def _tc_prep(x, W, degp_t):
    return pl.pallas_call(
        _prep_body,
        grid=(_NB,),
        in_specs=[
            pl.BlockSpec((_BS, _D), lambda i: (i, 0)),
            pl.BlockSpec((_D, _D), lambda i: (0, 0)),
            pl.BlockSpec((_BS, _NC), lambda i: (i, 0)),
        ],
        out_specs=[
            pl.BlockSpec((_BS, _D), lambda i: (i, 0)),
            pl.BlockSpec((_BS, 1), lambda i: (i, 0)),
        ],
        out_shape=[
            jax.ShapeDtypeStruct((_N, _D), jnp.float32),
            jax.ShapeDtypeStruct((_N, 1), jnp.float32),
        ],
    )(x, W, degp_t)


def _z_stats_phase(h_ref, u_ref, inv_ref, z_ref, st_ref, i):
    z = (h_ref[0] + h_ref[1] + u_ref[...]) * inv_ref[...]
    z_ref[pl.ds(i * _BS, _BS)] = z
    s1 = jnp.sum(z, axis=0, keepdims=True)
    s2 = jnp.sum(z * z, axis=0, keepdims=True)
    blk = jnp.concatenate([s1, s2], axis=0)

    @pl.when(i == 0)
    def _():
        st_ref[...] = blk

    @pl.when(i != 0)
    def _():
        st_ref[...] += blk


def _bn_relu_phase(z_ref, st_ref, g_ref, be_ref, i):
    z = z_ref[pl.ds(i * _BS, _BS)]
    mean = st_ref[0:1] * (1.0 / _N)
    var = st_ref[1:2] * (1.0 / _N) - mean * mean
    a = lax.rsqrt(var + 1e-5) * g_ref[...]
    return jnp.maximum((z - mean) * a + be_ref[...], 0.0)


def _layer_body(h_ref, u_ref, inv_ref, g_ref, be_ref, w_ref, out_ref,
                z_ref, st_ref):
    p = pl.program_id(0)
    i = pl.program_id(1)

    @pl.when(p == 0)
    def _():
        _z_stats_phase(h_ref, u_ref, inv_ref, z_ref, st_ref, i)

    @pl.when(p == 1)
    def _():
        y = _bn_relu_phase(z_ref, st_ref, g_ref, be_ref, i)
        out_ref[...] = jnp.dot(
            y, w_ref[...], preferred_element_type=jnp.float32) * inv_ref[...]


def _tc_layer(h, u, inv, gamma, beta, Wn):
    return pl.pallas_call(
        _layer_body,
        grid=(2, _NB),
        in_specs=[
            pl.BlockSpec((_NC, _BS, _D), lambda p, i: (0, i * (1 - p), 0)),
            pl.BlockSpec((_BS, _D), lambda p, i: (i * (1 - p), 0)),
            pl.BlockSpec((_BS, 1), lambda p, i: (i, 0)),
            pl.BlockSpec((1, _D), lambda p, i: (0, 0)),
            pl.BlockSpec((1, _D), lambda p, i: (0, 0)),
            pl.BlockSpec((_D, _D), lambda p, i: (0, 0)),
        ],
        out_specs=pl.BlockSpec((_BS, _D), lambda p, i: (i * p, 0)),
        out_shape=jax.ShapeDtypeStruct((_N, _D), jnp.float32),
        scratch_shapes=[
            pltpu.VMEM((_N, _D), jnp.float32),
            pltpu.VMEM((2, _D), jnp.float32),
        ],
    )(h, u, inv, gamma.reshape(1, _D), beta.reshape(1, _D), Wn)


def _poolf_body(h_ref, u_ref, inv_ref, g_ref, be_ref, b_ref, w_ref, bo_ref,
                out_ref, z_ref, st_ref, acc_ref):
    p = pl.program_id(0)
    i = pl.program_id(1)

    @pl.when(p == 0)
    def _():
        _z_stats_phase(h_ref, u_ref, inv_ref, z_ref, st_ref, i)

    @pl.when(p == 1)
    def _():
        y = _bn_relu_phase(z_ref, st_ref, g_ref, be_ref, i)
        seg = lax.broadcasted_iota(jnp.int32, (_BS, _G), 1)
        onehot = (b_ref[...] == seg).astype(jnp.float32)
        part = lax.dot_general(onehot, y, (((0,), (0,)), ((), ())),
                               preferred_element_type=jnp.float32)

        @pl.when(i == 0)
        def _():
            acc_ref[...] = part

        @pl.when(i != 0)
        def _():
            acc_ref[...] += part

        @pl.when(i == _NB - 1)
        def _():
            out_ref[...] = jnp.dot(
                acc_ref[...], w_ref[...],
                preferred_element_type=jnp.float32) + bo_ref[...]


def _tc_poolf(h, u, inv, gamma, beta, batch, Wout, bout):
    return pl.pallas_call(
        _poolf_body,
        grid=(2, _NB),
        in_specs=[
            pl.BlockSpec((_NC, _BS, _D), lambda p, i: (0, i * (1 - p), 0)),
            pl.BlockSpec((_BS, _D), lambda p, i: (i * (1 - p), 0)),
            pl.BlockSpec((_BS, 1), lambda p, i: (i, 0)),
            pl.BlockSpec((1, _D), lambda p, i: (0, 0)),
            pl.BlockSpec((1, _D), lambda p, i: (0, 0)),
            pl.BlockSpec((_BS, 1), lambda p, i: (i, 0)),
            pl.BlockSpec((_D, _D), lambda p, i: (0, 0)),
            pl.BlockSpec((1, _D), lambda p, i: (0, 0)),
        ],
        out_specs=pl.BlockSpec((_G, _D), lambda p, i: (0, 0)),
        out_shape=jax.ShapeDtypeStruct((_G, _D), jnp.float32),
        scratch_shapes=[
            pltpu.VMEM((_N, _D), jnp.float32),
            pltpu.VMEM((2, _D), jnp.float32),
            pltpu.VMEM((_G, _D), jnp.float32),
        ],
    )(h, u, inv, gamma.reshape(1, _D), beta.reshape(1, _D),
      batch, Wout, bout.reshape(1, _D))


def kernel(x, edge_index, batch, W1, b1, gamma1, beta1, W2, b2, gamma2,
           beta2, W3, b3, gamma3, beta3, Wout, bout):
    x = x.astype(jnp.float32)
    src2 = edge_index[0].reshape(_NW, _EP)
    dst2 = edge_index[1].reshape(_NW, _EP)
    dst3d = edge_index[1].reshape(_NW, _NCHD, _CD)

    degp = _sc_degree(dst3d)
    u, inv = _tc_prep(x, W1, degp.T)

    for gamma, beta, Wn in ((gamma1, beta1, W2), (gamma2, beta2, W3)):
        h = _sc_propagate(u, src2, dst2)
        u = _tc_layer(h, u, inv, gamma, beta, Wn)

    h = _sc_propagate(u, src2, dst2)
    return _tc_poolf(h, u, inv, gamma3, beta3, batch.reshape(_N, 1),
                     Wout, bout)
